# Initial kernel scaffold; baseline (speedup 1.0000x reference)
import jax
import jax.numpy as jnp
from jax.experimental import pallas as pl


def kernel(x, edge_index, batch, W1l, W1r, b1, W2l, W2r, b2, W3l, W3r, b3):
    raise NotImplementedError("write your pallas kernel here")


# speedup vs baseline: 2.5300x; 2.5300x over previous
import functools

import jax
import jax.numpy as jnp
from jax import lax
from jax.experimental import pallas as pl
from jax.experimental.pallas import tpu as pltpu
from jax.experimental.pallas import tpu_sc as plsc

N = 10000
E = 320000
D = 128
G = 16

NP = 10240
DUMP = NP - 1
NC = 2
NS = 16
NW = NC * NS
CH = 64
IB = 8
NBLK = 20
EPW = CH * IB * NBLK
EP = EPW * NW
CW = 128

ROW_BLK = 1024
GRID_TC = NP // ROW_BLK


_SC_MESH = plsc.VectorSubcoreMesh(core_axis_name="c", subcore_axis_name="s",
                                  num_cores=NC, num_subcores=NS)


def _sc_aggregate_body(h_hbm, src_hbm, dst_hbm, zfeat_hbm, feat_out,
                       acc, src_v, dst_v, rows_v):
    cid = lax.axis_index("c")
    tid = lax.axis_index("s")
    wid = cid * NS + tid

    @pl.when(tid == 0)
    def _():
        pltpu.sync_copy(zfeat_hbm, acc)

    plsc.subcore_barrier()

    def block(b, carry):
        pltpu.sync_copy(src_hbm.at[wid, b], src_v)
        pltpu.sync_copy(dst_hbm.at[wid, b], dst_v)
        for k in range(IB):
            pltpu.sync_copy(h_hbm.at[src_v.at[k]], rows_v)
            pltpu.sync_copy(rows_v, acc.at[dst_v.at[k]], add=True)
        return carry

    lax.fori_loop(0, NBLK, block, 0)
    plsc.subcore_barrier()

    @pl.when(tid == 0)
    def _():
        pltpu.sync_copy(acc, feat_out.at[cid])


_sc_aggregate = pl.kernel(
    _sc_aggregate_body,
    out_type=[jax.ShapeDtypeStruct((NC, NP, D), jnp.float32)],
    mesh=_SC_MESH,
    scratch_types=[
        pltpu.VMEM_SHARED((NP, D), jnp.float32),
        pltpu.VMEM((IB, CH), jnp.int32),
        pltpu.VMEM((IB, CH), jnp.int32),
        pltpu.VMEM((CH, D), jnp.float32),
    ],
)


def _sc_count_body(dst_hbm, zcnt_hbm, ones_hbm, cnt_out,
                   cacc, dst_v, ones_v):
    cid = lax.axis_index("c")
    tid = lax.axis_index("s")
    wid = cid * NS + tid

    @pl.when(tid == 0)
    def _():
        pltpu.sync_copy(zcnt_hbm, cacc)

    pltpu.sync_copy(ones_hbm, ones_v)
    plsc.subcore_barrier()

    def block(b, carry):
        pltpu.sync_copy(dst_hbm.at[wid, b], dst_v)
        for k in range(IB):
            pltpu.sync_copy(ones_v, cacc.at[dst_v.at[k]], add=True)
        return carry

    lax.fori_loop(0, NBLK, block, 0)
    plsc.subcore_barrier()

    @pl.when(tid == 0)
    def _():
        pltpu.sync_copy(cacc, cnt_out.at[cid])


_sc_count = pl.kernel(
    _sc_count_body,
    out_type=[jax.ShapeDtypeStruct((NC, NP, CW), jnp.float32)],
    mesh=_SC_MESH,
    scratch_types=[
        pltpu.VMEM_SHARED((NP, CW), jnp.float32),
        pltpu.VMEM((IB, CH), jnp.int32),
        pltpu.VMEM((CH, CW), jnp.float32),
    ],
)


def _tc_layer_body(relu, a0, a1, x, c0, c1, wl, wr, b, out):
    cnt = jnp.maximum(c0[...] + c1[...], 1.0)
    mean = (a0[...] + a1[...]) / cnt
    z = (jnp.dot(mean, wl[...], preferred_element_type=jnp.float32)
         + jnp.dot(x[...], wr[...], preferred_element_type=jnp.float32)
         + b[...])
    if relu:
        z = jnp.maximum(z, 0.0)
    nrm = jnp.maximum(jnp.sqrt(jnp.sum(z * z, axis=1, keepdims=True)), 1e-12)
    out[...] = z / nrm


def _tc_layer(a0, a1, x, c0, c1, wl, wr, b, relu):
    blk = lambda r, c: pl.BlockSpec((r, c), lambda i: (i, 0))
    full = pl.BlockSpec((D, D), lambda i: (0, 0))
    return pl.pallas_call(
        functools.partial(_tc_layer_body, relu),
        grid=(GRID_TC,),
        in_specs=[blk(ROW_BLK, D), blk(ROW_BLK, D), blk(ROW_BLK, D),
                  blk(ROW_BLK, 1), blk(ROW_BLK, 1),
                  full, full, pl.BlockSpec((1, D), lambda i: (0, 0))],
        out_specs=blk(ROW_BLK, D),
        out_shape=jax.ShapeDtypeStruct((NP, D), jnp.float32),
    )(a0, a1, x, c0, c1, wl, wr, b)


def _tc_final_body(a0, a1, x, c0, c1, wl, wr, b, batch, out, psum, pcnt):
    i = pl.program_id(0)

    cnt = jnp.maximum(c0[...] + c1[...], 1.0)
    mean = (a0[...] + a1[...]) / cnt
    z = (jnp.dot(mean, wl[...], preferred_element_type=jnp.float32)
         + jnp.dot(x[...], wr[...], preferred_element_type=jnp.float32)
         + b[...])
    nrm = jnp.maximum(jnp.sqrt(jnp.sum(z * z, axis=1, keepdims=True)), 1e-12)
    h = z / nrm

    gid = lax.broadcasted_iota(jnp.int32, (G, 1), 0)
    onehot = (batch[0] == gid).astype(jnp.float32)

    @pl.when(i == 0)
    def _():
        psum[...] = jnp.zeros_like(psum)
        pcnt[...] = jnp.zeros_like(pcnt)

    psum[...] += jnp.dot(onehot, h, preferred_element_type=jnp.float32)
    pcnt[...] += jnp.sum(onehot, axis=1, keepdims=True)

    @pl.when(i == GRID_TC - 1)
    def _():
        out[...] = psum[...] / jnp.maximum(pcnt[...], 1.0)


def _tc_final(a0, a1, x, c0, c1, wl, wr, b, batch3):
    blk = lambda r, c: pl.BlockSpec((r, c), lambda i: (i, 0))
    full = pl.BlockSpec((D, D), lambda i: (0, 0))
    return pl.pallas_call(
        _tc_final_body,
        grid=(GRID_TC,),
        in_specs=[blk(ROW_BLK, D), blk(ROW_BLK, D), blk(ROW_BLK, D),
                  blk(ROW_BLK, 1), blk(ROW_BLK, 1),
                  full, full, pl.BlockSpec((1, D), lambda i: (0, 0)),
                  pl.BlockSpec((1, 1, ROW_BLK), lambda i: (i, 0, 0))],
        out_specs=pl.BlockSpec((G, D), lambda i: (0, 0)),
        out_shape=jax.ShapeDtypeStruct((G, D), jnp.float32),
        scratch_shapes=[pltpu.VMEM((G, D), jnp.float32),
                        pltpu.VMEM((G, 1), jnp.float32)],
    )(a0, a1, x, c0, c1, wl, wr, b, batch3)


def kernel(x, edge_index, batch, W1l, W1r, b1, W2l, W2r, b2, W3l, W3r, b3):
    xp = jnp.pad(x, ((0, NP - N), (0, 0)))
    src = jnp.pad(edge_index[0], (0, EP - E))
    dst = jnp.pad(edge_index[1], (0, EP - E), constant_values=DUMP)
    src3 = src.reshape(NW, NBLK, IB, CH)
    dst3 = dst.reshape(NW, NBLK, IB, CH)
    batch3 = jnp.pad(batch, (0, NP - N), constant_values=G).reshape(
        GRID_TC, 1, ROW_BLK)
    zfeat = jnp.zeros((NP, D), jnp.float32)
    zcnt = jnp.zeros((NP, CW), jnp.float32)
    ones = jnp.ones((CH, CW), jnp.float32)

    (cnt,) = _sc_count(dst3, zcnt, ones)
    (feat,) = _sc_aggregate(xp, src3, dst3, zfeat)
    c0 = cnt[0, :, :1]
    c1 = cnt[1, :, :1]
    h = _tc_layer(feat[0], feat[1], xp, c0, c1, W1l, W1r,
                  b1.reshape(1, D), relu=True)

    (feat,) = _sc_aggregate(h, src3, dst3, zfeat)
    h = _tc_layer(feat[0], feat[1], h, c0, c1, W2l, W2r,
                  b2.reshape(1, D), relu=True)

    (feat,) = _sc_aggregate(h, src3, dst3, zfeat)
    return _tc_final(feat[0], feat[1], h, c0, c1, W3l, W3r,
                     b3.reshape(1, D), batch3)


# speedup vs baseline: 5.7149x; 2.2589x over previous
import functools

import jax
import jax.numpy as jnp
from jax import lax
from jax.experimental import pallas as pl
from jax.experimental.pallas import tpu as pltpu
from jax.experimental.pallas import tpu_sc as plsc

N = 10000
E = 320000
D = 128
G = 16

NP = 10240
DUMP = NP - 1
NC = 2
NS = 16
NW = NC * NS
CH = 64
IB = 8
NBLK = 20
EPW = CH * IB * NBLK
EP = EPW * NW
CW = 128

ROW_BLK = 1024
GRID_TC = NP // ROW_BLK


_SC_MESH = plsc.VectorSubcoreMesh(core_axis_name="c", subcore_axis_name="s",
                                  num_cores=NC, num_subcores=NS)


def _sc_aggregate_body(h_hbm, src_hbm, dst_hbm, zfeat_hbm, feat_out,
                       acc, src_v, dst_v, rows_v):
    cid = lax.axis_index("c")
    tid = lax.axis_index("s")
    wid = cid * NS + tid

    @pl.when(tid == 0)
    def _():
        pltpu.sync_copy(zfeat_hbm, acc)

    plsc.subcore_barrier()

    def block(b, carry):
        pltpu.sync_copy(src_hbm.at[wid, b], src_v)
        pltpu.sync_copy(dst_hbm.at[wid, b], dst_v)
        for k in range(IB):
            pltpu.sync_copy(h_hbm.at[src_v.at[k]], rows_v)
            pltpu.sync_copy(rows_v, acc.at[dst_v.at[k]], add=True)
        return carry

    lax.fori_loop(0, NBLK, block, 0)
    plsc.subcore_barrier()

    @pl.when(tid == 0)
    def _():
        pltpu.sync_copy(acc, feat_out.at[cid])


_sc_aggregate = pl.kernel(
    _sc_aggregate_body,
    out_type=[jax.ShapeDtypeStruct((NC, NP, D), jnp.float32)],
    mesh=_SC_MESH,
    scratch_types=[
        pltpu.VMEM_SHARED((NP, D), jnp.float32),
        pltpu.VMEM((IB, CH), jnp.int32),
        pltpu.VMEM((IB, CH), jnp.int32),
        pltpu.VMEM((CH, D), jnp.float32),
    ],
)


def _sc_count_body(dst_hbm, zcnt_hbm, ones_hbm, cnt_out,
                   cacc, dst_v, ones_v):
    cid = lax.axis_index("c")
    tid = lax.axis_index("s")
    wid = cid * NS + tid

    @pl.when(tid == 0)
    def _():
        pltpu.sync_copy(zcnt_hbm, cacc)

    pltpu.sync_copy(ones_hbm, ones_v)
    plsc.subcore_barrier()

    def block(b, carry):
        pltpu.sync_copy(dst_hbm.at[wid, b], dst_v)
        for k in range(IB):
            pltpu.sync_copy(ones_v, cacc.at[dst_v.at[k]], add=True)
        return carry

    lax.fori_loop(0, NBLK, block, 0)
    plsc.subcore_barrier()

    @pl.when(tid == 0)
    def _():
        pltpu.sync_copy(cacc, cnt_out.at[cid])


_sc_count = pl.kernel(
    _sc_count_body,
    out_type=[jax.ShapeDtypeStruct((NC, NP, CW), jnp.float32)],
    mesh=_SC_MESH,
    scratch_types=[
        pltpu.VMEM_SHARED((NP, CW), jnp.float32),
        pltpu.VMEM((IB, CH), jnp.int32),
        pltpu.VMEM((CH, CW), jnp.float32),
    ],
)


def _tc_layer_body(relu, a0, a1, x, c0, c1, wl, wr, b, out):
    cnt = jnp.maximum(c0[...] + c1[...], 1.0)
    mean = (a0[...] + a1[...]) / cnt
    z = (jnp.dot(mean, wl[...], preferred_element_type=jnp.float32)
         + jnp.dot(x[...], wr[...], preferred_element_type=jnp.float32)
         + b[...])
    if relu:
        z = jnp.maximum(z, 0.0)
    nrm = jnp.maximum(jnp.sqrt(jnp.sum(z * z, axis=1, keepdims=True)), 1e-12)
    out[...] = z / nrm


def _tc_layer(a0, a1, x, c0, c1, wl, wr, b, relu):
    blk = lambda r, c: pl.BlockSpec((r, c), lambda i: (i, 0))
    full = pl.BlockSpec((D, D), lambda i: (0, 0))
    return pl.pallas_call(
        functools.partial(_tc_layer_body, relu),
        grid=(GRID_TC,),
        in_specs=[blk(ROW_BLK, D), blk(ROW_BLK, D), blk(ROW_BLK, D),
                  blk(ROW_BLK, 1), blk(ROW_BLK, 1),
                  full, full, pl.BlockSpec((1, D), lambda i: (0, 0))],
        out_specs=blk(ROW_BLK, D),
        out_shape=jax.ShapeDtypeStruct((NP, D), jnp.float32),
    )(a0, a1, x, c0, c1, wl, wr, b)


def _tc_final_body(a0, a1, x, c0, c1, wl, wr, b, batch, out, psum, pcnt):
    i = pl.program_id(0)

    cnt = jnp.maximum(c0[...] + c1[...], 1.0)
    mean = (a0[...] + a1[...]) / cnt
    z = (jnp.dot(mean, wl[...], preferred_element_type=jnp.float32)
         + jnp.dot(x[...], wr[...], preferred_element_type=jnp.float32)
         + b[...])
    nrm = jnp.maximum(jnp.sqrt(jnp.sum(z * z, axis=1, keepdims=True)), 1e-12)
    h = z / nrm

    gid = lax.broadcasted_iota(jnp.int32, (G, 1), 0)
    onehot = (batch[0] == gid).astype(jnp.float32)

    @pl.when(i == 0)
    def _():
        psum[...] = jnp.zeros_like(psum)
        pcnt[...] = jnp.zeros_like(pcnt)

    psum[...] += jnp.dot(onehot, h, preferred_element_type=jnp.float32)
    pcnt[...] += jnp.sum(onehot, axis=1, keepdims=True)

    @pl.when(i == GRID_TC - 1)
    def _():
        out[...] = psum[...] / jnp.maximum(pcnt[...], 1.0)


def _tc_final(a0, a1, x, c0, c1, wl, wr, b, batch3):
    blk = lambda r, c: pl.BlockSpec((r, c), lambda i: (i, 0))
    full = pl.BlockSpec((D, D), lambda i: (0, 0))
    return pl.pallas_call(
        _tc_final_body,
        grid=(GRID_TC,),
        in_specs=[blk(ROW_BLK, D), blk(ROW_BLK, D), blk(ROW_BLK, D),
                  blk(ROW_BLK, 1), blk(ROW_BLK, 1),
                  full, full, pl.BlockSpec((1, D), lambda i: (0, 0)),
                  pl.BlockSpec((1, 1, ROW_BLK), lambda i: (i, 0, 0))],
        out_specs=pl.BlockSpec((G, D), lambda i: (0, 0)),
        out_shape=jax.ShapeDtypeStruct((G, D), jnp.float32),
        scratch_shapes=[pltpu.VMEM((G, D), jnp.float32),
                        pltpu.VMEM((G, 1), jnp.float32)],
    )(a0, a1, x, c0, c1, wl, wr, b, batch3)


def kernel(x, edge_index, batch, W1l, W1r, b1, W2l, W2r, b2, W3l, W3r, b3):
    xp = jnp.pad(x, ((0, NP - N), (0, 0)))
    pad_i = jnp.arange(EP - E, dtype=jnp.int32)
    src = jnp.concatenate([edge_index[0], pad_i % N])
    dst = jnp.concatenate([edge_index[1], N + pad_i % (NP - N)])
    src3 = src.reshape(NW, NBLK, IB, CH)
    dst3 = dst.reshape(NW, NBLK, IB, CH)
    batch3 = jnp.pad(batch, (0, NP - N), constant_values=G).reshape(
        GRID_TC, 1, ROW_BLK)
    zfeat = jnp.zeros((NP, D), jnp.float32)
    zcnt = jnp.zeros((NP, CW), jnp.float32)
    ones = jnp.ones((CH, CW), jnp.float32)

    (cnt,) = _sc_count(dst3, zcnt, ones)
    (feat,) = _sc_aggregate(xp, src3, dst3, zfeat)
    c0 = cnt[0, :, :1]
    c1 = cnt[1, :, :1]
    h = _tc_layer(feat[0], feat[1], xp, c0, c1, W1l, W1r,
                  b1.reshape(1, D), relu=True)

    (feat,) = _sc_aggregate(h, src3, dst3, zfeat)
    h = _tc_layer(feat[0], feat[1], h, c0, c1, W2l, W2r,
                  b2.reshape(1, D), relu=True)

    (feat,) = _sc_aggregate(h, src3, dst3, zfeat)
    return _tc_final(feat[0], feat[1], h, c0, c1, W3l, W3r,
                     b3.reshape(1, D), batch3)


# speedup vs baseline: 7.9191x; 1.3857x over previous
import functools

import jax
import jax.numpy as jnp
from jax import lax
from jax.experimental import pallas as pl
from jax.experimental.pallas import tpu as pltpu
from jax.experimental.pallas import tpu_sc as plsc

N = 10000
E = 320000
D = 128
G = 16

NP = 10240
DUMP = NP - 1
NC = 2
NS = 16
NW = NC * NS
CH = 64
IB = 8
NBLK = 20
EPW = CH * IB * NBLK
EP = EPW * NW
CW = 128

ROW_BLK = 1024
GRID_TC = NP // ROW_BLK


_SC_MESH = plsc.VectorSubcoreMesh(core_axis_name="c", subcore_axis_name="s",
                                  num_cores=NC, num_subcores=NS)


def _sc_aggregate_body(h_hbm, src_hbm, dst_hbm, zfeat_hbm, feat_out,
                       acc, src_v, dst_v, rows0, rows1, sem0, sem1):
    cid = lax.axis_index("c")
    tid = lax.axis_index("s")
    wid = cid * NS + tid

    @pl.when(tid == 0)
    def _():
        pltpu.sync_copy(zfeat_hbm, acc)

    plsc.subcore_barrier()

    bufs = (rows0, rows1)
    sems = (sem0, sem1)

    def block(b, carry):
        pltpu.sync_copy(src_hbm.at[wid, b], src_v)
        pltpu.sync_copy(dst_hbm.at[wid, b], dst_v)
        cps = [None, None]
        cps[0] = pltpu.async_copy(h_hbm.at[src_v.at[0]], bufs[0], sems[0])
        for k in range(IB):
            if k + 1 < IB:
                cps[(k + 1) % 2] = pltpu.async_copy(
                    h_hbm.at[src_v.at[k + 1]], bufs[(k + 1) % 2],
                    sems[(k + 1) % 2])
            cps[k % 2].wait()
            pltpu.sync_copy(bufs[k % 2], acc.at[dst_v.at[k]], add=True)
        return carry

    lax.fori_loop(0, NBLK, block, 0)
    plsc.subcore_barrier()

    @pl.when(tid == 0)
    def _():
        pltpu.sync_copy(acc, feat_out.at[cid])


_sc_aggregate = pl.kernel(
    _sc_aggregate_body,
    out_type=[jax.ShapeDtypeStruct((NC, NP, D), jnp.float32)],
    mesh=_SC_MESH,
    scratch_types=[
        pltpu.VMEM_SHARED((NP, D), jnp.float32),
        pltpu.VMEM((IB, CH), jnp.int32),
        pltpu.VMEM((IB, CH), jnp.int32),
        pltpu.VMEM((CH, D), jnp.float32),
        pltpu.VMEM((CH, D), jnp.float32),
        pltpu.SemaphoreType.DMA,
        pltpu.SemaphoreType.DMA,
    ],
)


def _sc_count_body(dst_hbm, zcnt_hbm, ones_hbm, cnt_out,
                   cacc, dst_v, ones_v):
    cid = lax.axis_index("c")
    tid = lax.axis_index("s")
    wid = cid * NS + tid

    @pl.when(tid == 0)
    def _():
        pltpu.sync_copy(zcnt_hbm, cacc)

    pltpu.sync_copy(ones_hbm, ones_v)
    plsc.subcore_barrier()

    def block(b, carry):
        pltpu.sync_copy(dst_hbm.at[wid, b], dst_v)
        for k in range(IB):
            pltpu.sync_copy(ones_v, cacc.at[dst_v.at[k]], add=True)
        return carry

    lax.fori_loop(0, NBLK, block, 0)
    plsc.subcore_barrier()

    @pl.when(tid == 0)
    def _():
        pltpu.sync_copy(cacc, cnt_out.at[cid])


_sc_count = pl.kernel(
    _sc_count_body,
    out_type=[jax.ShapeDtypeStruct((NC, NP, CW), jnp.float32)],
    mesh=_SC_MESH,
    scratch_types=[
        pltpu.VMEM_SHARED((NP, CW), jnp.float32),
        pltpu.VMEM((IB, CH), jnp.int32),
        pltpu.VMEM((CH, CW), jnp.float32),
    ],
)


def _tc_layer_body(relu, a0, a1, x, c0, c1, wl, wr, b, out):
    cnt = jnp.maximum(c0[...] + c1[...], 1.0)
    mean = (a0[...] + a1[...]) / cnt
    z = (jnp.dot(mean, wl[...], preferred_element_type=jnp.float32)
         + jnp.dot(x[...], wr[...], preferred_element_type=jnp.float32)
         + b[...])
    if relu:
        z = jnp.maximum(z, 0.0)
    nrm = jnp.maximum(jnp.sqrt(jnp.sum(z * z, axis=1, keepdims=True)), 1e-12)
    out[...] = z / nrm


def _tc_layer(a0, a1, x, c0, c1, wl, wr, b, relu):
    blk = lambda r, c: pl.BlockSpec((r, c), lambda i: (i, 0))
    full = pl.BlockSpec((D, D), lambda i: (0, 0))
    return pl.pallas_call(
        functools.partial(_tc_layer_body, relu),
        grid=(GRID_TC,),
        in_specs=[blk(ROW_BLK, D), blk(ROW_BLK, D), blk(ROW_BLK, D),
                  blk(ROW_BLK, 1), blk(ROW_BLK, 1),
                  full, full, pl.BlockSpec((1, D), lambda i: (0, 0))],
        out_specs=blk(ROW_BLK, D),
        out_shape=jax.ShapeDtypeStruct((NP, D), jnp.float32),
    )(a0, a1, x, c0, c1, wl, wr, b)


def _tc_final_body(a0, a1, x, c0, c1, wl, wr, b, batch, out, psum, pcnt):
    i = pl.program_id(0)

    cnt = jnp.maximum(c0[...] + c1[...], 1.0)
    mean = (a0[...] + a1[...]) / cnt
    z = (jnp.dot(mean, wl[...], preferred_element_type=jnp.float32)
         + jnp.dot(x[...], wr[...], preferred_element_type=jnp.float32)
         + b[...])
    nrm = jnp.maximum(jnp.sqrt(jnp.sum(z * z, axis=1, keepdims=True)), 1e-12)
    h = z / nrm

    gid = lax.broadcasted_iota(jnp.int32, (G, 1), 0)
    onehot = (batch[0] == gid).astype(jnp.float32)

    @pl.when(i == 0)
    def _():
        psum[...] = jnp.zeros_like(psum)
        pcnt[...] = jnp.zeros_like(pcnt)

    psum[...] += jnp.dot(onehot, h, preferred_element_type=jnp.float32)
    pcnt[...] += jnp.sum(onehot, axis=1, keepdims=True)

    @pl.when(i == GRID_TC - 1)
    def _():
        out[...] = psum[...] / jnp.maximum(pcnt[...], 1.0)


def _tc_final(a0, a1, x, c0, c1, wl, wr, b, batch3):
    blk = lambda r, c: pl.BlockSpec((r, c), lambda i: (i, 0))
    full = pl.BlockSpec((D, D), lambda i: (0, 0))
    return pl.pallas_call(
        _tc_final_body,
        grid=(GRID_TC,),
        in_specs=[blk(ROW_BLK, D), blk(ROW_BLK, D), blk(ROW_BLK, D),
                  blk(ROW_BLK, 1), blk(ROW_BLK, 1),
                  full, full, pl.BlockSpec((1, D), lambda i: (0, 0)),
                  pl.BlockSpec((1, 1, ROW_BLK), lambda i: (i, 0, 0))],
        out_specs=pl.BlockSpec((G, D), lambda i: (0, 0)),
        out_shape=jax.ShapeDtypeStruct((G, D), jnp.float32),
        scratch_shapes=[pltpu.VMEM((G, D), jnp.float32),
                        pltpu.VMEM((G, 1), jnp.float32)],
    )(a0, a1, x, c0, c1, wl, wr, b, batch3)


def kernel(x, edge_index, batch, W1l, W1r, b1, W2l, W2r, b2, W3l, W3r, b3):
    xp = jnp.pad(x, ((0, NP - N), (0, 0)))
    pad_i = jnp.arange(EP - E, dtype=jnp.int32)
    src = jnp.concatenate([edge_index[0], pad_i % N])
    dst = jnp.concatenate([edge_index[1], N + pad_i % (NP - N)])
    src3 = src.reshape(NW, NBLK, IB, CH)
    dst3 = dst.reshape(NW, NBLK, IB, CH)
    batch3 = jnp.pad(batch, (0, NP - N), constant_values=G).reshape(
        GRID_TC, 1, ROW_BLK)
    zfeat = jnp.zeros((NP, D), jnp.float32)
    zcnt = jnp.zeros((NP, CW), jnp.float32)
    ones = jnp.ones((CH, CW), jnp.float32)

    (cnt,) = _sc_count(dst3, zcnt, ones)
    (feat,) = _sc_aggregate(xp, src3, dst3, zfeat)
    c0 = cnt[0, :, :1]
    c1 = cnt[1, :, :1]
    h = _tc_layer(feat[0], feat[1], xp, c0, c1, W1l, W1r,
                  b1.reshape(1, D), relu=True)

    (feat,) = _sc_aggregate(h, src3, dst3, zfeat)
    h = _tc_layer(feat[0], feat[1], h, c0, c1, W2l, W2r,
                  b2.reshape(1, D), relu=True)

    (feat,) = _sc_aggregate(h, src3, dst3, zfeat)
    return _tc_final(feat[0], feat[1], h, c0, c1, W3l, W3r,
                     b3.reshape(1, D), batch3)


# speedup vs baseline: 7.9350x; 1.0020x over previous
import functools

import jax
import jax.numpy as jnp
from jax import lax
from jax.experimental import pallas as pl
from jax.experimental.pallas import tpu as pltpu
from jax.experimental.pallas import tpu_sc as plsc

N = 10000
E = 320000
D = 128
G = 16

NP = 10240
DUMP = NP - 1
NC = 2
NS = 16
NW = NC * NS
CH = 64
IB = 8
NBLK = 20
EPW = CH * IB * NBLK
EP = EPW * NW
CW = 128

ROW_BLK = 1024
GRID_TC = NP // ROW_BLK


_SC_MESH = plsc.VectorSubcoreMesh(core_axis_name="c", subcore_axis_name="s",
                                  num_cores=NC, num_subcores=NS)


def _sc_aggregate_body(h_hbm, src_hbm, dst_hbm, zfeat_hbm, feat_out,
                       acc, src_v, dst_v, rows0, rows1, sem0, sem1):
    cid = lax.axis_index("c")
    tid = lax.axis_index("s")
    wid = cid * NS + tid

    rsl = pl.ds(tid * (NP // NS), NP // NS)
    pltpu.sync_copy(zfeat_hbm.at[rsl], acc.at[rsl])

    plsc.subcore_barrier()

    bufs = (rows0, rows1)
    sems = (sem0, sem1)

    def block(b, carry):
        pltpu.sync_copy(src_hbm.at[wid, b], src_v)
        pltpu.sync_copy(dst_hbm.at[wid, b], dst_v)
        cps = [None, None]
        cps[0] = pltpu.async_copy(h_hbm.at[src_v.at[0]], bufs[0], sems[0])
        for k in range(IB):
            if k + 1 < IB:
                cps[(k + 1) % 2] = pltpu.async_copy(
                    h_hbm.at[src_v.at[k + 1]], bufs[(k + 1) % 2],
                    sems[(k + 1) % 2])
            cps[k % 2].wait()
            pltpu.sync_copy(bufs[k % 2], acc.at[dst_v.at[k]], add=True)
        return carry

    lax.fori_loop(0, NBLK, block, 0)
    plsc.subcore_barrier()

    pltpu.sync_copy(acc.at[rsl], feat_out.at[cid, rsl])


_sc_aggregate = pl.kernel(
    _sc_aggregate_body,
    out_type=[jax.ShapeDtypeStruct((NC, NP, D), jnp.float32)],
    mesh=_SC_MESH,
    scratch_types=[
        pltpu.VMEM_SHARED((NP, D), jnp.float32),
        pltpu.VMEM((IB, CH), jnp.int32),
        pltpu.VMEM((IB, CH), jnp.int32),
        pltpu.VMEM((CH, D), jnp.float32),
        pltpu.VMEM((CH, D), jnp.float32),
        pltpu.SemaphoreType.DMA,
        pltpu.SemaphoreType.DMA,
    ],
)


def _sc_count_body(dst_hbm, zcnt_hbm, ones_hbm, cnt_out,
                   cacc, dst_v, ones_v):
    cid = lax.axis_index("c")
    tid = lax.axis_index("s")
    wid = cid * NS + tid

    rsl = pl.ds(tid * (NP // NS), NP // NS)
    pltpu.sync_copy(zcnt_hbm.at[rsl], cacc.at[rsl])

    pltpu.sync_copy(ones_hbm, ones_v)
    plsc.subcore_barrier()

    def block(b, carry):
        pltpu.sync_copy(dst_hbm.at[wid, b], dst_v)
        for k in range(IB):
            pltpu.sync_copy(ones_v, cacc.at[dst_v.at[k]], add=True)
        return carry

    lax.fori_loop(0, NBLK, block, 0)
    plsc.subcore_barrier()

    pltpu.sync_copy(cacc.at[rsl], cnt_out.at[cid, rsl])


_sc_count = pl.kernel(
    _sc_count_body,
    out_type=[jax.ShapeDtypeStruct((NC, NP, CW), jnp.float32)],
    mesh=_SC_MESH,
    scratch_types=[
        pltpu.VMEM_SHARED((NP, CW), jnp.float32),
        pltpu.VMEM((IB, CH), jnp.int32),
        pltpu.VMEM((CH, CW), jnp.float32),
    ],
)


def _tc_layer_body(relu, a0, a1, x, c0, c1, wl, wr, b, out):
    cnt = jnp.maximum(c0[...] + c1[...], 1.0)
    mean = (a0[...] + a1[...]) / cnt
    z = (jnp.dot(mean, wl[...], preferred_element_type=jnp.float32)
         + jnp.dot(x[...], wr[...], preferred_element_type=jnp.float32)
         + b[...])
    if relu:
        z = jnp.maximum(z, 0.0)
    nrm = jnp.maximum(jnp.sqrt(jnp.sum(z * z, axis=1, keepdims=True)), 1e-12)
    out[...] = z / nrm


def _tc_layer(a0, a1, x, c0, c1, wl, wr, b, relu):
    blk = lambda r, c: pl.BlockSpec((r, c), lambda i: (i, 0))
    full = pl.BlockSpec((D, D), lambda i: (0, 0))
    return pl.pallas_call(
        functools.partial(_tc_layer_body, relu),
        grid=(GRID_TC,),
        in_specs=[blk(ROW_BLK, D), blk(ROW_BLK, D), blk(ROW_BLK, D),
                  blk(ROW_BLK, 1), blk(ROW_BLK, 1),
                  full, full, pl.BlockSpec((1, D), lambda i: (0, 0))],
        out_specs=blk(ROW_BLK, D),
        out_shape=jax.ShapeDtypeStruct((NP, D), jnp.float32),
    )(a0, a1, x, c0, c1, wl, wr, b)


def _tc_final_body(a0, a1, x, c0, c1, wl, wr, b, batch, out, psum, pcnt):
    i = pl.program_id(0)

    cnt = jnp.maximum(c0[...] + c1[...], 1.0)
    mean = (a0[...] + a1[...]) / cnt
    z = (jnp.dot(mean, wl[...], preferred_element_type=jnp.float32)
         + jnp.dot(x[...], wr[...], preferred_element_type=jnp.float32)
         + b[...])
    nrm = jnp.maximum(jnp.sqrt(jnp.sum(z * z, axis=1, keepdims=True)), 1e-12)
    h = z / nrm

    gid = lax.broadcasted_iota(jnp.int32, (G, 1), 0)
    onehot = (batch[0] == gid).astype(jnp.float32)

    @pl.when(i == 0)
    def _():
        psum[...] = jnp.zeros_like(psum)
        pcnt[...] = jnp.zeros_like(pcnt)

    psum[...] += jnp.dot(onehot, h, preferred_element_type=jnp.float32)
    pcnt[...] += jnp.sum(onehot, axis=1, keepdims=True)

    @pl.when(i == GRID_TC - 1)
    def _():
        out[...] = psum[...] / jnp.maximum(pcnt[...], 1.0)


def _tc_final(a0, a1, x, c0, c1, wl, wr, b, batch3):
    blk = lambda r, c: pl.BlockSpec((r, c), lambda i: (i, 0))
    full = pl.BlockSpec((D, D), lambda i: (0, 0))
    return pl.pallas_call(
        _tc_final_body,
        grid=(GRID_TC,),
        in_specs=[blk(ROW_BLK, D), blk(ROW_BLK, D), blk(ROW_BLK, D),
                  blk(ROW_BLK, 1), blk(ROW_BLK, 1),
                  full, full, pl.BlockSpec((1, D), lambda i: (0, 0)),
                  pl.BlockSpec((1, 1, ROW_BLK), lambda i: (i, 0, 0))],
        out_specs=pl.BlockSpec((G, D), lambda i: (0, 0)),
        out_shape=jax.ShapeDtypeStruct((G, D), jnp.float32),
        scratch_shapes=[pltpu.VMEM((G, D), jnp.float32),
                        pltpu.VMEM((G, 1), jnp.float32)],
    )(a0, a1, x, c0, c1, wl, wr, b, batch3)


def kernel(x, edge_index, batch, W1l, W1r, b1, W2l, W2r, b2, W3l, W3r, b3):
    xp = jnp.pad(x, ((0, NP - N), (0, 0)))
    pad_i = jnp.arange(EP - E, dtype=jnp.int32)
    src = jnp.concatenate([edge_index[0], pad_i % N])
    dst = jnp.concatenate([edge_index[1], N + pad_i % (NP - N)])
    src3 = src.reshape(NW, NBLK, IB, CH)
    dst3 = dst.reshape(NW, NBLK, IB, CH)
    batch3 = jnp.pad(batch, (0, NP - N), constant_values=G).reshape(
        GRID_TC, 1, ROW_BLK)
    zfeat = jnp.zeros((NP, D), jnp.float32)
    zcnt = jnp.zeros((NP, CW), jnp.float32)
    ones = jnp.ones((CH, CW), jnp.float32)

    (cnt,) = _sc_count(dst3, zcnt, ones)
    (feat,) = _sc_aggregate(xp, src3, dst3, zfeat)
    c0 = cnt[0, :, :1]
    c1 = cnt[1, :, :1]
    h = _tc_layer(feat[0], feat[1], xp, c0, c1, W1l, W1r,
                  b1.reshape(1, D), relu=True)

    (feat,) = _sc_aggregate(h, src3, dst3, zfeat)
    h = _tc_layer(feat[0], feat[1], h, c0, c1, W2l, W2r,
                  b2.reshape(1, D), relu=True)

    (feat,) = _sc_aggregate(h, src3, dst3, zfeat)
    return _tc_final(feat[0], feat[1], h, c0, c1, W3l, W3r,
                     b3.reshape(1, D), batch3)


# speedup vs baseline: 8.6209x; 1.0864x over previous
import functools

import jax
import jax.numpy as jnp
from jax import lax
from jax.experimental import pallas as pl
from jax.experimental.pallas import tpu as pltpu
from jax.experimental.pallas import tpu_sc as plsc

N = 10000
E = 320000
D = 128
G = 16

NP = 10240
DUMP = NP - 1
NC = 2
NS = 16
NW = NC * NS
CH = 64
IB = 8
NBLK = 20
EPW = CH * IB * NBLK
EP = EPW * NW
CW = 128

ROW_BLK = 1024
GRID_TC = NP // ROW_BLK


_SC_MESH = plsc.VectorSubcoreMesh(core_axis_name="c", subcore_axis_name="s",
                                  num_cores=NC, num_subcores=NS)


def _sc_aggregate_body(h_hbm, idx_hbm, zfeat_hbm, feat_out,
                       acc, idxa, idxb, rows0, rows1,
                       sem0, sem1, isema, isemb):
    cid = lax.axis_index("c")
    tid = lax.axis_index("s")
    wid = cid * NS + tid

    rsl = pl.ds(tid * (NP // NS), NP // NS)
    pltpu.sync_copy(zfeat_hbm.at[rsl], acc.at[rsl])

    plsc.subcore_barrier()

    bufs = (rows0, rows1)
    sems = (sem0, sem1)

    def process(idx_buf):
        cps = [None, None]
        cps[0] = pltpu.async_copy(h_hbm.at[idx_buf.at[0, 0]], bufs[0],
                                  sems[0])
        for k in range(IB):
            if k + 1 < IB:
                cps[(k + 1) % 2] = pltpu.async_copy(
                    h_hbm.at[idx_buf.at[0, k + 1]], bufs[(k + 1) % 2],
                    sems[(k + 1) % 2])
            cps[k % 2].wait()
            pltpu.sync_copy(bufs[k % 2], acc.at[idx_buf.at[1, k]], add=True)

    def idx_wait(buf, sem):
        pltpu.make_async_copy(idx_hbm.at[wid, 0], buf, sem).wait()

    pltpu.async_copy(idx_hbm.at[wid, 0], idxa, isema)

    def pair(j, carry):
        b0 = 2 * j
        idx_wait(idxa, isema)
        pltpu.async_copy(idx_hbm.at[wid, b0 + 1], idxb, isemb)
        process(idxa)
        idx_wait(idxb, isemb)
        pltpu.async_copy(idx_hbm.at[wid, jnp.minimum(b0 + 2, NBLK - 1)],
                         idxa, isema)
        process(idxb)
        return carry

    lax.fori_loop(0, NBLK // 2, pair, 0)
    idx_wait(idxa, isema)
    plsc.subcore_barrier()

    pltpu.sync_copy(acc.at[rsl], feat_out.at[cid, rsl])


_sc_aggregate = pl.kernel(
    _sc_aggregate_body,
    out_type=[jax.ShapeDtypeStruct((NC, NP, D), jnp.float32)],
    mesh=_SC_MESH,
    scratch_types=[
        pltpu.VMEM_SHARED((NP, D), jnp.float32),
        pltpu.VMEM((2, IB, CH), jnp.int32),
        pltpu.VMEM((2, IB, CH), jnp.int32),
        pltpu.VMEM((CH, D), jnp.float32),
        pltpu.VMEM((CH, D), jnp.float32),
        pltpu.SemaphoreType.DMA,
        pltpu.SemaphoreType.DMA,
        pltpu.SemaphoreType.DMA,
        pltpu.SemaphoreType.DMA,
    ],
)


def _sc_count_body(dst_hbm, zcnt_hbm, ones_hbm, cnt_out,
                   cacc, dst_v, ones_v):
    cid = lax.axis_index("c")
    tid = lax.axis_index("s")
    wid = cid * NS + tid

    rsl = pl.ds(tid * (NP // NS), NP // NS)
    pltpu.sync_copy(zcnt_hbm.at[rsl], cacc.at[rsl])

    pltpu.sync_copy(ones_hbm, ones_v)
    plsc.subcore_barrier()

    def block(b, carry):
        pltpu.sync_copy(dst_hbm.at[wid, b], dst_v)
        for k in range(IB):
            pltpu.sync_copy(ones_v, cacc.at[dst_v.at[k]], add=True)
        return carry

    lax.fori_loop(0, NBLK, block, 0)
    plsc.subcore_barrier()

    pltpu.sync_copy(cacc.at[rsl], cnt_out.at[cid, rsl])


_sc_count = pl.kernel(
    _sc_count_body,
    out_type=[jax.ShapeDtypeStruct((NC, NP, CW), jnp.float32)],
    mesh=_SC_MESH,
    scratch_types=[
        pltpu.VMEM_SHARED((NP, CW), jnp.float32),
        pltpu.VMEM((IB, CH), jnp.int32),
        pltpu.VMEM((CH, CW), jnp.float32),
    ],
)


def _tc_layer_body(relu, a0, a1, x, c0, c1, wl, wr, b, out):
    cnt = jnp.maximum(c0[...] + c1[...], 1.0)
    mean = (a0[...] + a1[...]) / cnt
    z = (jnp.dot(mean, wl[...], preferred_element_type=jnp.float32)
         + jnp.dot(x[...], wr[...], preferred_element_type=jnp.float32)
         + b[...])
    if relu:
        z = jnp.maximum(z, 0.0)
    nrm = jnp.maximum(jnp.sqrt(jnp.sum(z * z, axis=1, keepdims=True)), 1e-12)
    out[...] = z / nrm


def _tc_layer(a0, a1, x, c0, c1, wl, wr, b, relu):
    blk = lambda r, c: pl.BlockSpec((r, c), lambda i: (i, 0))
    full = pl.BlockSpec((D, D), lambda i: (0, 0))
    return pl.pallas_call(
        functools.partial(_tc_layer_body, relu),
        grid=(GRID_TC,),
        in_specs=[blk(ROW_BLK, D), blk(ROW_BLK, D), blk(ROW_BLK, D),
                  blk(ROW_BLK, 1), blk(ROW_BLK, 1),
                  full, full, pl.BlockSpec((1, D), lambda i: (0, 0))],
        out_specs=blk(ROW_BLK, D),
        out_shape=jax.ShapeDtypeStruct((NP, D), jnp.float32),
    )(a0, a1, x, c0, c1, wl, wr, b)


def _tc_final_body(a0, a1, x, c0, c1, wl, wr, b, batch, out, psum, pcnt):
    i = pl.program_id(0)

    cnt = jnp.maximum(c0[...] + c1[...], 1.0)
    mean = (a0[...] + a1[...]) / cnt
    z = (jnp.dot(mean, wl[...], preferred_element_type=jnp.float32)
         + jnp.dot(x[...], wr[...], preferred_element_type=jnp.float32)
         + b[...])
    nrm = jnp.maximum(jnp.sqrt(jnp.sum(z * z, axis=1, keepdims=True)), 1e-12)
    h = z / nrm

    gid = lax.broadcasted_iota(jnp.int32, (G, 1), 0)
    onehot = (batch[0] == gid).astype(jnp.float32)

    @pl.when(i == 0)
    def _():
        psum[...] = jnp.zeros_like(psum)
        pcnt[...] = jnp.zeros_like(pcnt)

    psum[...] += jnp.dot(onehot, h, preferred_element_type=jnp.float32)
    pcnt[...] += jnp.sum(onehot, axis=1, keepdims=True)

    @pl.when(i == GRID_TC - 1)
    def _():
        out[...] = psum[...] / jnp.maximum(pcnt[...], 1.0)


def _tc_final(a0, a1, x, c0, c1, wl, wr, b, batch3):
    blk = lambda r, c: pl.BlockSpec((r, c), lambda i: (i, 0))
    full = pl.BlockSpec((D, D), lambda i: (0, 0))
    return pl.pallas_call(
        _tc_final_body,
        grid=(GRID_TC,),
        in_specs=[blk(ROW_BLK, D), blk(ROW_BLK, D), blk(ROW_BLK, D),
                  blk(ROW_BLK, 1), blk(ROW_BLK, 1),
                  full, full, pl.BlockSpec((1, D), lambda i: (0, 0)),
                  pl.BlockSpec((1, 1, ROW_BLK), lambda i: (i, 0, 0))],
        out_specs=pl.BlockSpec((G, D), lambda i: (0, 0)),
        out_shape=jax.ShapeDtypeStruct((G, D), jnp.float32),
        scratch_shapes=[pltpu.VMEM((G, D), jnp.float32),
                        pltpu.VMEM((G, 1), jnp.float32)],
    )(a0, a1, x, c0, c1, wl, wr, b, batch3)


def kernel(x, edge_index, batch, W1l, W1r, b1, W2l, W2r, b2, W3l, W3r, b3):
    xp = jnp.pad(x, ((0, NP - N), (0, 0)))
    pad_i = jnp.arange(EP - E, dtype=jnp.int32)
    src = jnp.concatenate([edge_index[0], pad_i % N])
    dst = jnp.concatenate([edge_index[1], N + pad_i % (NP - N)])
    src3 = src.reshape(NW, NBLK, IB, CH)
    dst3 = dst.reshape(NW, NBLK, IB, CH)
    idx3 = jnp.stack([src3, dst3], axis=2)
    batch3 = jnp.pad(batch, (0, NP - N), constant_values=G).reshape(
        GRID_TC, 1, ROW_BLK)
    zfeat = jnp.zeros((NP, D), jnp.float32)
    zcnt = jnp.zeros((NP, CW), jnp.float32)
    ones = jnp.ones((CH, CW), jnp.float32)

    (cnt,) = _sc_count(dst3, zcnt, ones)
    (feat,) = _sc_aggregate(xp, idx3, zfeat)
    c0 = cnt[0, :, :1]
    c1 = cnt[1, :, :1]
    h = _tc_layer(feat[0], feat[1], xp, c0, c1, W1l, W1r,
                  b1.reshape(1, D), relu=True)

    (feat,) = _sc_aggregate(h, idx3, zfeat)
    h = _tc_layer(feat[0], feat[1], h, c0, c1, W2l, W2r,
                  b2.reshape(1, D), relu=True)

    (feat,) = _sc_aggregate(h, idx3, zfeat)
    return _tc_final(feat[0], feat[1], h, c0, c1, W3l, W3r,
                     b3.reshape(1, D), batch3)
